# Initial kernel scaffold; baseline (speedup 1.0000x reference)
import jax
import jax.numpy as jnp
from jax.experimental import pallas as pl


def kernel(input_ids, target_logits, num_previously_accepted):
    raise NotImplementedError("write your pallas kernel here")


# speedup vs baseline: 1.8837x; 1.8837x over previous
import functools

import jax
import jax.numpy as jnp
from jax import lax
from jax.experimental import pallas as pl
from jax.experimental.pallas import tpu as pltpu


def _body(prev_ref, ids_ref, logits_ref, draft_ref, counts_ref, last_ref):
    b = pl.program_id(0)
    S = ids_ref.shape[2]
    V = logits_ref.shape[2]

    logits = logits_ref[0]
    mx = jnp.max(logits, axis=-1, keepdims=True)
    vio = lax.broadcasted_iota(jnp.int32, (S, V), 1)
    greedy = jnp.min(jnp.where(logits == mx, vio, jnp.int32(V)), axis=-1)
    greedy = greedy.astype(jnp.int32).reshape(1, S)
    ids = ids_ref[0]

    p = prev_ref[b]

    matches = (greedy[:, : S - 1] == ids[:, 1:])
    m_idx = lax.broadcasted_iota(jnp.int32, (1, S - 1), 1)
    mism = jnp.where(jnp.logical_and(m_idx >= p - 1, jnp.logical_not(matches)),
                     m_idx, S - 1)
    first = jnp.min(mism)
    num_newly = jnp.maximum(first - (p - 1), 0)
    na = p + num_newly

    k_idx = lax.broadcasted_iota(jnp.int32, (1, S), 1)
    bonus = jnp.sum(jnp.where(k_idx == na - 1, greedy, 0))
    shifted = jnp.concatenate(
        [ids[:, 1:], jnp.zeros((1, 1), dtype=ids.dtype)], axis=1)
    draft = jnp.where(k_idx < na - 1, shifted,
                      jnp.where(k_idx == na - 1, bonus, 0)).astype(ids.dtype)
    draft_ref[0] = draft

    c_idx = lax.broadcasted_iota(jnp.int32, (1, 8), 1)
    counts_ref[0] = jnp.where(c_idx == 0, num_newly,
                              jnp.where(c_idx == 1, na, 0))

    last_ref[0] = logits_ref[0, pl.ds(na - 1, 1), :]


def kernel(input_ids, target_logits, num_previously_accepted):
    B, S = input_ids.shape
    V = target_logits.shape[2]
    ids3 = input_ids.reshape(B, 1, S)
    prev = num_previously_accepted.astype(jnp.int32)

    grid = (B,)
    draft3, counts3, last3 = pl.pallas_call(
        _body,
        grid=grid,
        in_specs=[
            pl.BlockSpec(memory_space=pltpu.SMEM),
            pl.BlockSpec((1, 1, S), lambda b: (b, 0, 0)),
            pl.BlockSpec((1, S, V), lambda b: (b, 0, 0)),
        ],
        out_specs=[
            pl.BlockSpec((1, 1, S), lambda b: (b, 0, 0)),
            pl.BlockSpec((1, 1, 8), lambda b: (b, 0, 0)),
            pl.BlockSpec((1, 1, V), lambda b: (b, 0, 0)),
        ],
        out_shape=[
            jax.ShapeDtypeStruct((B, 1, S), jnp.int32),
            jax.ShapeDtypeStruct((B, 1, 8), jnp.int32),
            jax.ShapeDtypeStruct((B, 1, V), jnp.float32),
        ],
    )(prev, ids3, target_logits)

    draft_input_ids = draft3.reshape(B, S).astype(input_ids.dtype)
    num_newly = counts3[:, 0, 0].astype(num_previously_accepted.dtype)
    num_accepted = counts3[:, 0, 1].astype(num_previously_accepted.dtype)
    return (draft_input_ids, num_newly, num_accepted, last3)
